# baseline (speedup 1.0000x reference)
import functools

import jax
import jax.numpy as jnp
from jax import lax
from jax.experimental import pallas as pl
from jax.experimental.pallas import tpu as pltpu
from jax.experimental.pallas import tpu_sc as plsc

B = 16384
D = 64
L = 16
NC = 2
NS = 16
NW = NC * NS
BPW = B // NW
CHUNK = 128
NCHUNK = BPW // CHUNK

_mesh = plsc.VectorSubcoreMesh(core_axis_name="c", subcore_axis_name="s")


@functools.partial(
    pl.kernel,
    out_type=jax.ShapeDtypeStruct((B,), jnp.float32),
    mesh=_mesh,
    compiler_params=pltpu.CompilerParams(
        needs_layout_passes=False, use_tc_tiling_on_sc=False),
    scratch_types=[
        pltpu.VMEM((NCHUNK, CHUNK), jnp.int32),
        pltpu.VMEM((NCHUNK, CHUNK), jnp.int32),
        pltpu.VMEM((CHUNK, D), jnp.float32),
        pltpu.VMEM((CHUNK, D), jnp.float32),
        pltpu.VMEM((BPW,), jnp.float32),
        pltpu.VMEM((L * L,), jnp.float32),
        pltpu.VMEM((D,), jnp.float32),
        pltpu.VMEM((L,), jnp.float32),
        pltpu.SemaphoreType.DMA,
        pltpu.SemaphoreType.DMA,
    ],
)
def _gmf_sc(uidx_hbm, vidx_hbm, ut_hbm, it_hbm, w_hbm, b_hbm, out_hbm,
            uidx_v, vidx_v, urows, vrows, outv, mat, wv, bv, sem_u, sem_v):
    wid = lax.axis_index("s") * NC + lax.axis_index("c")
    base = wid * BPW

    pltpu.sync_copy(uidx_hbm.at[wid], uidx_v)
    pltpu.sync_copy(vidx_hbm.at[wid], vidx_v)
    pltpu.sync_copy(w_hbm, wv)
    pltpu.sync_copy(b_hbm, bv)

    w = [wv[pl.ds(c * L, L)] for c in range(D // L)]
    bvec = bv[pl.ds(0, L)]
    col_base = lax.iota(jnp.int32, L) * L
    idxcol = [col_base + l for l in range(L)]

    for j in range(NCHUNK):
        cu = pltpu.async_copy(ut_hbm.at[uidx_v.at[j]], urows, sem_u)
        cv = pltpu.async_copy(it_hbm.at[vidx_v.at[j]], vrows, sem_v)
        cu.wait()
        cv.wait()

        def group_body(g, carry, j=j):
            i0 = g * L
            for r in range(L):
                acc = (urows[i0 + r, pl.ds(0, L)]
                       * vrows[i0 + r, pl.ds(0, L)]) * w[0]
                for c in range(1, D // L):
                    acc += (urows[i0 + r, pl.ds(c * L, L)]
                            * vrows[i0 + r, pl.ds(c * L, L)]) * w[c]
                mat[pl.ds(r * L, L)] = acc
            colsum = bvec
            for l in range(L):
                colsum = colsum + plsc.load_gather(mat, [idxcol[l]])
            outv[pl.ds(j * CHUNK + i0, L)] = 1.0 / (1.0 + jnp.exp(-colsum))
            return carry

        lax.fori_loop(0, CHUNK // L, group_body, 0)

    pltpu.sync_copy(outv, out_hbm.at[pl.ds(base, BPW)])


def kernel(input, user_table, item_table, W, b):
    idx = input.astype(jnp.int32)
    uidx = idx[:, 0].reshape(NW, NCHUNK, CHUNK)
    vidx = idx[:, 1].reshape(NW, NCHUNK, CHUNK)
    w_flat = W.reshape(D)
    b_vec = jnp.broadcast_to(b, (L,))
    return _gmf_sc(uidx, vidx, user_table, item_table, w_flat, b_vec)
